# Initial kernel scaffold; baseline (speedup 1.0000x reference)
import jax
import jax.numpy as jnp
from jax.experimental import pallas as pl


def kernel(x, W, reputation_scores, expert_loads, expert_counts, total_routing_decisions):
    raise NotImplementedError("write your pallas kernel here")


# speedup vs baseline: 1.0166x; 1.0166x over previous
import functools

import jax
import jax.numpy as jnp
from jax.experimental import pallas as pl
from jax.experimental.pallas import tpu as pltpu

HIDDEN = 2048
NUM_EXPERTS = 16
TOP_K = 2
BETA = 0.1
GAMMA = 0.1
EXPLORATION_C = 0.1
LOAD_EMA_ALPHA = 0.9

TB = 1024


def _router_block(x_ref, w_ref, bias_ref, wout_ref, iout_ref, aux_ref,
                  acc_ref):
    step = pl.program_id(0)
    nsteps = pl.num_programs(0)

    @pl.when(step == 0)
    def _():
        acc_ref[...] = jnp.zeros_like(acc_ref)

    logits = jax.lax.dot_general(
        x_ref[...], w_ref[...],
        dimension_numbers=(((1,), (1,)), ((), ())),
        preferred_element_type=jnp.float32)
    logits = logits + bias_ref[...]

    iota_e = jax.lax.broadcasted_iota(jnp.int32, (TB, NUM_EXPERTS), 1)

    m1 = jnp.max(logits, axis=1, keepdims=True)
    i1 = jnp.argmax(logits, axis=1).reshape(TB, 1)
    masked = jnp.where(iota_e == i1, -jnp.inf, logits)
    m2 = jnp.max(masked, axis=1, keepdims=True)
    i2 = jnp.argmax(masked, axis=1).reshape(TB, 1)

    e2 = jnp.exp(m2 - m1)
    w1 = 1.0 / (1.0 + e2)
    w2 = 1.0 - w1

    pair = jax.lax.broadcasted_iota(jnp.int32, (TB, TOP_K), 1)
    wout_ref[...] = jnp.where(pair == 0, w1, w2)
    iout_ref[...] = jnp.where(pair == 0, i1, i2)

    p = jnp.exp(logits - m1)
    probs = p / jnp.sum(p, axis=1, keepdims=True)
    prob_sum = jnp.sum(probs, axis=0, keepdims=True)

    gate = ((iota_e == i1).astype(jnp.float32)
            + (iota_e == i2).astype(jnp.float32))
    cnt_sum = jnp.sum(gate, axis=0, keepdims=True)

    acc_ref[0:1, :] += prob_sum
    acc_ref[1:2, :] += cnt_sum

    @pl.when(step == nsteps - 1)
    def _():
        total_t = jnp.float32(TB) * nsteps
        aux = (jnp.sum(acc_ref[0:1, :] * acc_ref[1:2, :])
               * NUM_EXPERTS / (total_t * total_t))
        aux_ref[0, 0] = aux


@functools.partial(jax.jit, static_argnames=())
def _router(x2, W, bias):
    T = x2.shape[0]
    grid = (T // TB,)
    wout, iout, aux = pl.pallas_call(
        _router_block,
        grid=grid,
        in_specs=[
            pl.BlockSpec((TB, HIDDEN), lambda i: (i, 0)),
            pl.BlockSpec((NUM_EXPERTS, HIDDEN), lambda i: (0, 0)),
            pl.BlockSpec((1, NUM_EXPERTS), lambda i: (0, 0)),
        ],
        out_specs=[
            pl.BlockSpec((TB, TOP_K), lambda i: (i, 0)),
            pl.BlockSpec((TB, TOP_K), lambda i: (i, 0)),
            pl.BlockSpec(memory_space=pltpu.SMEM),
        ],
        out_shape=[
            jax.ShapeDtypeStruct((T, TOP_K), jnp.float32),
            jax.ShapeDtypeStruct((T, TOP_K), jnp.int32),
            jax.ShapeDtypeStruct((1, 1), jnp.float32),
        ],
        scratch_shapes=[pltpu.VMEM((8, NUM_EXPERTS), jnp.float32)],
    )(x2, W, bias)
    return wout, iout, aux


def kernel(x, W, reputation_scores, expert_loads, expert_counts,
           total_routing_decisions):
    B, S, H = x.shape
    x2 = x.reshape(-1, H)
    updated_loads = (LOAD_EMA_ALPHA * expert_loads
                     + (1.0 - LOAD_EMA_ALPHA) * expert_loads)
    exploration = EXPLORATION_C * jnp.sqrt(
        jnp.log(total_routing_decisions + 1.0) / (expert_counts + 1e-10))
    bias = (BETA * reputation_scores - GAMMA * updated_loads
            + exploration).reshape(1, NUM_EXPERTS).astype(jnp.float32)

    wout, iout, aux = _router(x2, W, bias)
    routing_weights = wout.reshape(B, S, TOP_K)
    expert_indices = iout.reshape(B, S, TOP_K)
    return routing_weights, expert_indices, aux.reshape(())
